# baseline (speedup 1.0000x reference)
import functools

import jax
import jax.numpy as jnp
from jax import lax
from jax.experimental import pallas as pl
from jax.experimental.pallas import tpu as pltpu
from jax.experimental.pallas import tpu_sc as plsc

_B = 16
_NS = 10475
_NO = 50000
_P = 128
_L = 16
_NC = _P // _L
_NPAIR = 2
_S_BASE = 4992
_S_DELTA = 8
_O_BASE = 19968
_O_DELTA = 32
_W = 256


def _sc_min_d2(smplx_t, object_t):
    info = plsc.get_sparse_core_info()
    nc, ns = info.num_cores, info.num_subcores
    nw = nc * ns
    mesh = plsc.VectorSubcoreMesh(core_axis_name="c", subcore_axis_name="s")

    @functools.partial(
        pl.kernel,
        mesh=mesh,
        out_type=jax.ShapeDtypeStruct((nw, _P), jnp.float32),
        scratch_types=[
            pltpu.VMEM((3, 8, _W), jnp.float32),
            pltpu.VMEM((3, 8, _W), jnp.float32),
            pltpu.VMEM((_P,), jnp.float32),
            pltpu.VMEM((_P,), jnp.float32),
            pltpu.VMEM((_P,), jnp.float32),
            pltpu.VMEM((_P,), jnp.float32),
            pltpu.VMEM((_P,), jnp.float32),
            pltpu.VMEM((_P,), jnp.float32),
            pltpu.SemaphoreType.DMA,
        ],
        compiler_params=pltpu.CompilerParams(needs_layout_passes=False),
    )
    def k(s_hbm, o_hbm, out_hbm, sv, ov, md2, abuf, bxb, byb, bzb, s2b, sem):
        wid = lax.axis_index("s") * nc + lax.axis_index("c")
        b = wid // _NPAIR
        p = wid % _NPAIR
        bb = (b // 8) * 8
        r = b - bb
        s_base = p * _S_BASE
        o_base = p * _O_BASE
        s_delta = p * _S_DELTA
        o_delta = p * _O_DELTA
        copies = []
        for kx in range(3):
            copies.append(pltpu.async_copy(
                s_hbm.at[kx, pl.ds(bb, 8), pl.ds(s_base, _W)], sv.at[kx], sem))
            copies.append(pltpu.async_copy(
                o_hbm.at[kx, pl.ds(bb, 8), pl.ds(o_base, _W)], ov.at[kx], sem))
        for cp in copies:
            cp.wait()

        iota = lax.iota(jnp.int32, _L)
        col0 = jnp.zeros((_L,), jnp.int32)
        col1 = col0 + 1
        col2 = col0 + 2
        rvec = jnp.full((_L,), r, jnp.int32)

        def oprep(c, _):
            orows = iota + (c * _L) + o_delta
            ox = plsc.load_gather(ov, [col0, rvec, orows])
            oy = plsc.load_gather(ov, [col1, rvec, orows])
            oz = plsc.load_gather(ov, [col2, rvec, orows])
            abuf[pl.ds(c * _L, _L)] = ox * ox + oy * oy + oz * oz
            bxb[pl.ds(c * _L, _L)] = ox * (-2.0)
            byb[pl.ds(c * _L, _L)] = oy * (-2.0)
            bzb[pl.ds(c * _L, _L)] = oz * (-2.0)
            return 0

        lax.fori_loop(0, _NC, oprep, 0)

        sxs, sys_, szs = [], [], []
        for c in range(_NC):
            srows = iota + (c * _L) + s_delta
            sx = plsc.load_gather(sv, [col0, rvec, srows])
            sy = plsc.load_gather(sv, [col1, rvec, srows])
            sz = plsc.load_gather(sv, [col2, rvec, srows])
            s2b[pl.ds(c * _L, _L)] = sx * sx + sy * sy + sz * sz
            sxs.append(sx)
            sys_.append(sy)
            szs.append(sz)

        inf = jnp.full((_L,), jnp.inf, jnp.float32)

        def body(j, ms):
            ji = jnp.full((_L,), j, jnp.int32)
            a = plsc.load_gather(abuf, [ji])
            bx = plsc.load_gather(bxb, [ji])
            by = plsc.load_gather(byb, [ji])
            bz = plsc.load_gather(bzb, [ji])
            out = []
            for c in range(_NC):
                t = a + bx * sxs[c] + by * sys_[c] + bz * szs[c]
                out.append(jnp.minimum(ms[c], t))
            return tuple(out)

        ms = lax.fori_loop(0, _P, body, (inf,) * _NC, unroll=2)
        for c in range(_NC):
            md2[pl.ds(c * _L, _L)] = ms[c] + s2b[pl.ds(c * _L, _L)]
        pltpu.sync_copy(md2, out_hbm.at[wid])

    return k(smplx_t, object_t)


def _tc_finish(md2):
    def body(x_ref, o_ref):
        d = jnp.sqrt(jnp.maximum(x_ref[...], 0.0))
        o_ref[0, 0] = jnp.sum(d) * (1.0 / (_B * _NPAIR * _P))

    out = pl.pallas_call(
        body,
        out_shape=jax.ShapeDtypeStruct((1, 1), jnp.float32),
        in_specs=[pl.BlockSpec(memory_space=pltpu.VMEM)],
        out_specs=pl.BlockSpec(memory_space=pltpu.SMEM),
    )(md2)
    return out[0, 0]


def kernel(smplx_v_centered, object_v_centered):
    st = jnp.transpose(smplx_v_centered, (2, 0, 1))
    ot = jnp.transpose(object_v_centered, (2, 0, 1))
    md2 = _sc_min_d2(st, ot)
    return _tc_finish(md2)


# speedup vs baseline: 1.0077x; 1.0077x over previous
import functools

import jax
import jax.numpy as jnp
from jax import lax
from jax.experimental import pallas as pl
from jax.experimental.pallas import tpu as pltpu
from jax.experimental.pallas import tpu_sc as plsc

_B = 16
_NS = 10475
_NO = 50000
_P = 128
_L = 16
_NC = _P // _L
_NPAIR = 2
_S_BASE = 4992
_S_DELTA = 8
_O_BASE = 19968
_O_DELTA = 32
_W = 256


def _sc_min_d2(smplx_t, object_t):
    info = plsc.get_sparse_core_info()
    nc, ns = info.num_cores, info.num_subcores
    nw = nc * ns
    mesh = plsc.VectorSubcoreMesh(core_axis_name="c", subcore_axis_name="s")

    @functools.partial(
        pl.kernel,
        mesh=mesh,
        out_type=jax.ShapeDtypeStruct((nw, _P), jnp.float32),
        scratch_types=[
            pltpu.VMEM((3, 8, _W), jnp.float32),
            pltpu.VMEM((3, 8, _W), jnp.float32),
            pltpu.VMEM((_P,), jnp.float32),
            pltpu.VMEM((_P,), jnp.float32),
            pltpu.VMEM((_P,), jnp.float32),
            pltpu.VMEM((_P,), jnp.float32),
            pltpu.VMEM((_P,), jnp.float32),
            pltpu.VMEM((_P,), jnp.float32),
            pltpu.SemaphoreType.DMA,
        ],
        compiler_params=pltpu.CompilerParams(needs_layout_passes=False),
    )
    def k(s_hbm, o_hbm, out_hbm, sv, ov, md2, abuf, bxb, byb, bzb, s2b, sem):
        wid = lax.axis_index("s") * nc + lax.axis_index("c")
        b = wid // _NPAIR
        p = wid % _NPAIR
        bb = (b // 8) * 8
        r = b - bb
        s_base = p * _S_BASE
        o_base = p * _O_BASE
        s_delta = p * _S_DELTA
        o_delta = p * _O_DELTA
        copies = []
        for kx in range(3):
            copies.append(pltpu.async_copy(
                s_hbm.at[kx, pl.ds(bb, 8), pl.ds(s_base, _W)], sv.at[kx], sem))
            copies.append(pltpu.async_copy(
                o_hbm.at[kx, pl.ds(bb, 8), pl.ds(o_base, _W)], ov.at[kx], sem))
        for cp in copies:
            cp.wait()

        iota = lax.iota(jnp.int32, _L)
        col0 = jnp.zeros((_L,), jnp.int32)
        col1 = col0 + 1
        col2 = col0 + 2
        rvec = jnp.full((_L,), r, jnp.int32)

        def oprep(c, _):
            orows = iota + (c * _L) + o_delta
            ox = plsc.load_gather(ov, [col0, rvec, orows])
            oy = plsc.load_gather(ov, [col1, rvec, orows])
            oz = plsc.load_gather(ov, [col2, rvec, orows])
            abuf[pl.ds(c * _L, _L)] = ox * ox + oy * oy + oz * oz
            bxb[pl.ds(c * _L, _L)] = ox * (-2.0)
            byb[pl.ds(c * _L, _L)] = oy * (-2.0)
            bzb[pl.ds(c * _L, _L)] = oz * (-2.0)
            return 0

        lax.fori_loop(0, _NC, oprep, 0)

        sxs, sys_, szs = [], [], []
        for c in range(_NC):
            srows = iota + (c * _L) + s_delta
            sx = plsc.load_gather(sv, [col0, rvec, srows])
            sy = plsc.load_gather(sv, [col1, rvec, srows])
            sz = plsc.load_gather(sv, [col2, rvec, srows])
            s2b[pl.ds(c * _L, _L)] = sx * sx + sy * sy + sz * sz
            sxs.append(sx)
            sys_.append(sy)
            szs.append(sz)

        inf = jnp.full((_L,), jnp.inf, jnp.float32)

        def body(j, ms):
            ji = jnp.full((_L,), j, jnp.int32)
            a = plsc.load_gather(abuf, [ji])
            bx = plsc.load_gather(bxb, [ji])
            by = plsc.load_gather(byb, [ji])
            bz = plsc.load_gather(bzb, [ji])
            out = []
            for c in range(_NC):
                t = a + bx * sxs[c] + by * sys_[c] + bz * szs[c]
                out.append(jnp.minimum(ms[c], t))
            return tuple(out)

        ms = lax.fori_loop(0, _P, body, (inf,) * _NC)
        for c in range(_NC):
            md2[pl.ds(c * _L, _L)] = ms[c] + s2b[pl.ds(c * _L, _L)]
        pltpu.sync_copy(md2, out_hbm.at[wid])

    return k(smplx_t, object_t)


def _tc_finish(md2):
    def body(x_ref, o_ref):
        d = jnp.sqrt(jnp.maximum(x_ref[...], 0.0))
        o_ref[0, 0] = jnp.sum(d) * (1.0 / (_B * _NPAIR * _P))

    out = pl.pallas_call(
        body,
        out_shape=jax.ShapeDtypeStruct((1, 1), jnp.float32),
        in_specs=[pl.BlockSpec(memory_space=pltpu.VMEM)],
        out_specs=pl.BlockSpec(memory_space=pltpu.SMEM),
    )(md2)
    return out[0, 0]


def kernel(smplx_v_centered, object_v_centered):
    st = jnp.transpose(smplx_v_centered, (2, 0, 1))
    ot = jnp.transpose(object_v_centered, (2, 0, 1))
    md2 = _sc_min_d2(st, ot)
    return _tc_finish(md2)
